# baseline (speedup 1.0000x reference)
import functools

import jax
import jax.numpy as jnp
import numpy as np
from jax import lax
from jax.experimental import pallas as pl
from jax.experimental.pallas import tpu as pltpu
from jax.experimental.pallas import tpu_sc as plsc

_RADIUS2 = 9.0
_NSAMPLE = 16
_H_RANGE, _W_RANGE = 5, 11
_KPAD = 64
_RH, _RW = 64, 2048


def _sc_kernel(xh, yh, zh, features, rv_flat, rows, cols, qx, qy, qz):
    C = features.shape[1] // 2
    M = rows.shape[0]
    NW = 32
    QPT = M // NW
    CO = 3 + C
    ROW = CO * _NSAMPLE
    G = 16
    NMT = QPT // 128

    mesh = plsc.VectorSubcoreMesh(core_axis_name="c", subcore_axis_name="s")

    @functools.partial(
        pl.kernel,
        out_type=jax.ShapeDtypeStruct((M * ROW,), jnp.float32),
        mesh=mesh,
        scratch_types=[
            pltpu.VMEM((QPT,), jnp.int32),
            pltpu.VMEM((QPT,), jnp.int32),
            pltpu.VMEM((QPT,), jnp.float32),
            pltpu.VMEM((QPT,), jnp.float32),
            pltpu.VMEM((QPT,), jnp.float32),
            pltpu.VMEM((8, 128), jnp.int32),
            pltpu.VMEM((8, 128), jnp.int32),
            pltpu.VMEM((8, 128), jnp.float32),
            pltpu.VMEM((8, 128), jnp.float32),
            pltpu.VMEM((8, 128), jnp.float32),
            pltpu.VMEM((80,), jnp.int32),
            pltpu.VMEM((80,), jnp.float32),
            pltpu.VMEM((80,), jnp.float32),
            pltpu.VMEM((80,), jnp.float32),
            pltpu.VMEM((1040,), jnp.int32),
            pltpu.VMEM((1040,), jnp.int32),
            pltpu.VMEM((1040,), jnp.int32),
            pltpu.VMEM((1040,), jnp.int32),
            pltpu.VMEM((144,), jnp.float32),
            pltpu.VMEM((2064,), jnp.float32),
            pltpu.VMEM((2064,), jnp.float32),
            pltpu.VMEM((2064,), jnp.float32),
            pltpu.VMEM((4, 32, 2 * C), jnp.float32),
            pltpu.VMEM((67 * 1024,), jnp.float32),
            pltpu.SemaphoreType.DMA,
            pltpu.SemaphoreType.DMA,
            pltpu.SemaphoreType.DMA,
            pltpu.SemaphoreType.DMA,
            pltpu.SemaphoreType.DMA,
        ],
        compiler_params=pltpu.CompilerParams(needs_layout_passes=False),
    )
    def k(xh_h, yh_h, zh_h, feat_h, rv_h, rows_h, cols_h, qx_h, qy_h, qz_h,
          out_h, rows_v, cols_v, qx_v, qy_v, qz_v, idxb, candb, cx, cy, cz,
          seli, selx, sely, selz, idsA, idsB, parA, parB, zfq,
          relx, rely, relz, featb, obuf, sem, fsem0, fsem1, fsem2, fsem3):
        wid = lax.axis_index("s") * 2 + lax.axis_index("c")
        base = wid * QPT
        pltpu.sync_copy(rows_h.at[pl.ds(base, QPT)], rows_v)
        pltpu.sync_copy(cols_h.at[pl.ds(base, QPT)], cols_v)
        pltpu.sync_copy(qx_h.at[pl.ds(base, QPT)], qx_v)
        pltpu.sync_copy(qy_h.at[pl.ds(base, QPT)], qy_v)
        pltpu.sync_copy(qz_h.at[pl.ds(base, QPT)], qz_v)

        iota = lax.iota(jnp.int32, 16)
        iota1024 = iota * 1024
        mA = iota < 8
        mB = iota >= 8
        fsems = [fsem0, fsem1, fsem2, fsem3]

        def window_rel(j):
            kv = iota + 16 * j
            hq = lax.shift_right_logical(kv * 94, 10)
            wq = kv - hq * _W_RANGE
            return hq - 2, wq - 5, kv

        def sel_query(qq, r, c, qxs, qys, qzs, q):
            cnt = jnp.int32(0)
            for j in range(_KPAD // 16):
                p = qq * _KPAD + j * 16
                rw, off = p // 128, p % 128
                cand = candb[rw, pl.ds(off, 16)]
                xs = cx[rw, pl.ds(off, 16)]
                ys = cy[rw, pl.ds(off, 16)]
                zs = cz[rw, pl.ds(off, 16)]
                hr, wr, kv = window_rel(j)
                rp = r + hr
                cp = c + wr
                ok = (rp >= 0) & (rp < _RH) & (cp >= 0) & (cp < _RW)
                if j == 3:
                    ok = ok & (kv < _H_RANGE * _W_RANGE)
                dx = xs - qxs
                dy = ys - qys
                dz = zs - qzs
                d2 = dx * dx + dy * dy + dz * dz
                inb = ok & (cand >= 0) & (d2 <= _RADIUS2)
                bi = jnp.where(inb, 1, 0)
                pos = plsc.cumsum(bi) - 1 + cnt
                plsc.store_scatter(seli, [pos], cand, mask=inb)
                plsc.store_scatter(selx, [pos], xs, mask=inb)
                plsc.store_scatter(sely, [pos], ys, mask=inb)
                plsc.store_scatter(selz, [pos], zs, mask=inb)
                cnt = cnt + jnp.sum(bi)

            nonempty = cnt > 0
            pos16 = jnp.where(iota < cnt, iota, 0)
            sid = plsc.load_gather(seli, [pos16])
            gx = plsc.load_gather(selx, [pos16])
            gy = plsc.load_gather(sely, [pos16])
            gz = plsc.load_gather(selz, [pos16])
            sid = jnp.where(nonempty, sid, 0)
            sidh = lax.shift_right_logical(sid, 1)
            sidp = sid & 1
            posA = q * 8 + iota
            plsc.store_scatter(idsA, [posA], sidh, mask=mA)
            plsc.store_scatter(idsB, [posA - 8], sidh, mask=mB)
            plsc.store_scatter(parA, [posA], sidp, mask=mA)
            plsc.store_scatter(parB, [posA - 8], sidp, mask=mB)
            zv = jnp.where(nonempty, jnp.float32(1.0), jnp.float32(0.0))
            zv = zv + iota * 0.0
            plsc.store_scatter(zfq, [iota * 0 + q], zv, mask=iota < 1)
            posr = q * 16 + iota
            plsc.store_scatter(relx, [posr],
                               jnp.where(nonempty, gx - qxs, 0.0))
            plsc.store_scatter(rely, [posr],
                               jnp.where(nonempty, gy - qys, 0.0))
            plsc.store_scatter(relz, [posr],
                               jnp.where(nonempty, gz - qzs, 0.0))

        def sel_group(mt, gi):
            start = mt * 128 + gi * G
            rows16 = rows_v[pl.ds(start, 16)]
            cols16 = cols_v[pl.ds(start, 16)]
            qx16 = qx_v[pl.ds(start, 16)]
            qy16 = qy_v[pl.ds(start, 16)]
            qz16 = qz_v[pl.ds(start, 16)]

            def sel_pair(rw):
                for qq in (2 * rw, 2 * rw + 1):
                    sel_query(qq, rows16[qq], cols16[qq],
                              qx16[qq], qy16[qq], qz16[qq], gi * G + qq)

            cand_cps = []
            for rw in range(8):
                for qq in (2 * rw, 2 * rw + 1):
                    r = rows16[qq]
                    c = cols16[qq]
                    for j in range(_KPAD // 16):
                        hr, wr, _ = window_rel(j)
                        rp = r + hr
                        cp = c + wr
                        rc = jnp.clip(rp, 0, _RH - 1)
                        cc = jnp.clip(cp, 0, _RW - 1)
                        p = qq * _KPAD + j * 16
                        idxb[p // 128, pl.ds(p % 128, 16)] = rc * _RW + cc
                cand_cps.append(pltpu.async_copy(
                    rv_h.at[idxb.at[rw]], candb.at[rw], sem))
            xyz_cps = {}
            for rw in range(8):
                cand_cps[rw].wait()
                xyz_cps[rw] = (
                    pltpu.async_copy(xh_h.at[candb.at[rw]], cx.at[rw],
                                     fsem0),
                    pltpu.async_copy(yh_h.at[candb.at[rw]], cy.at[rw],
                                     fsem0),
                    pltpu.async_copy(zh_h.at[candb.at[rw]], cz.at[rw],
                                     fsem0),
                )
                if rw >= 1:
                    for cp in xyz_cps[rw - 1]:
                        cp.wait()
                    sel_pair(rw - 1)
            for cp in xyz_cps[7]:
                cp.wait()
            sel_pair(7)

        def transpose_query(qq, buf, q, st, par, zq):
            parv = par[pl.ds(q * 8, 16)]
            rvx = relx[pl.ds(q * 16, 16)]
            rvy = rely[pl.ds(q * 16, 16)]
            rvz = relz[pl.ds(q * 16, 16)]
            if st == 0:
                mask, s8v = mA, iota
            else:
                mask, s8v = mB, iota - 8
            posr = s8v * 128 + q
            plsc.store_scatter(obuf, [posr], rvx, mask=mask)
            plsc.store_scatter(obuf, [posr + 1024], rvy, mask=mask)
            plsc.store_scatter(obuf, [posr + 2048], rvz, mask=mask)
            for s8 in range(8):
                off = pl.multiple_of(parv[s8] * C, C)
                frow = (qq % 4) * 8 + s8
                sbase = 3 * 1024 + s8 * 128 + q
                for cb in range(C // 16):
                    vec = featb[buf, frow, pl.ds(off + cb * 16, 16)]
                    posv = iota1024 + (sbase + cb * 16 * 1024)
                    plsc.store_scatter(obuf, [posv], vec * zq)

        def write_pass(mt, st):
            ids = idsA if st == 0 else idsB
            par = parA if st == 0 else parB

            def gi_body(gi, carry):
                gbase = gi * 128
                cp = pltpu.async_copy(
                    feat_h.at[ids.at[pl.ds(gbase, 32)]], featb.at[0],
                    fsems[0])
                for h in range(4):
                    nxt = None
                    if h < 3:
                        nxt = pltpu.async_copy(
                            feat_h.at[ids.at[pl.ds(gbase + (h + 1) * 32,
                                                   32)]],
                            featb.at[(h + 1) % 2], fsems[(h + 1) % 2])
                    cp.wait()
                    zv16 = zfq[pl.ds(gi * 16, 16)]
                    for qq in range(h * 4, h * 4 + 4):
                        transpose_query(qq, (h % 2), gi * G + qq, st, par,
                                        zv16[qq])
                    cp = nxt
                return carry

            lax.fori_loop(0, 128 // G, gi_body, jnp.int32(0))
            mtg = wid * NMT + mt
            cps = [pltpu.async_copy(
                obuf.at[pl.ds(ch * 1024, 1024)],
                out_h.at[pl.ds(((2 * ch + st) * 128 + mtg) * 1024, 1024)],
                sem)
                for ch in range(CO)]
            for cp in cps:
                cp.wait()

        def mt_body(mt, carry):
            def sel_gi(gi, c):
                sel_group(mt, gi)
                return c

            lax.fori_loop(0, 128 // G, sel_gi, jnp.int32(0))
            write_pass(mt, 0)
            write_pass(mt, 1)
            return carry

        lax.fori_loop(0, NMT, mt_body, jnp.int32(0))

    return k(xh, yh, zh, features, rv_flat, rows, cols, qx, qy, qz)


def _pair_rows_tc(features):
    N, C = features.shape
    BR = 8000
    grid = N // BR

    def body(x_ref, o_ref):
        o_ref[:, :C] = x_ref[0::2, :]
        o_ref[:, C:] = x_ref[1::2, :]

    return pl.pallas_call(
        body,
        grid=(grid,),
        in_specs=[pl.BlockSpec((BR, C), lambda i: (i, 0))],
        out_specs=pl.BlockSpec((BR // 2, 2 * C), lambda i: (i, 0)),
        out_shape=jax.ShapeDtypeStruct((N // 2, 2 * C), features.dtype),
    )(features)


def kernel(xyz, features, query_rv_xyz, query_rv_coords, rv_map):
    M = query_rv_xyz.shape[0]
    C = features.shape[1]
    features = _pair_rows_tc(features)
    rv_flat = rv_map.reshape(-1)
    rows = query_rv_coords[:, 1].astype(jnp.int32)
    cols = query_rv_coords[:, 2].astype(jnp.int32)
    xh = xyz[:, 0]
    yh = xyz[:, 1]
    zh = xyz[:, 2]
    qx = query_rv_xyz[:, 0]
    qy = query_rv_xyz[:, 1]
    qz = query_rv_xyz[:, 2]
    out = _sc_kernel(xh, yh, zh, features, rv_flat, rows, cols, qx, qy, qz)
    out = out.reshape(3 + C, 2, M // 128, 8, 128)
    out = out.transpose(2, 4, 0, 1, 3)
    return out.reshape(M, 3 + C, _NSAMPLE)


# speedup vs baseline: 1.0287x; 1.0287x over previous
import functools

import jax
import jax.numpy as jnp
import numpy as np
from jax import lax
from jax.experimental import pallas as pl
from jax.experimental.pallas import tpu as pltpu
from jax.experimental.pallas import tpu_sc as plsc

_RADIUS2 = 9.0
_NSAMPLE = 16
_H_RANGE, _W_RANGE = 5, 11
_KPAD = 64
_RH, _RW = 64, 2048


def _sc_kernel(xh, yh, zh, features, rv_flat, rows, cols, qx, qy, qz):
    C = features.shape[1] // 2
    M = rows.shape[0]
    NW = 32
    QPT = M // NW
    CO = 3 + C
    ROW = CO * _NSAMPLE
    G = 16
    NMT = QPT // 128

    mesh = plsc.VectorSubcoreMesh(core_axis_name="c", subcore_axis_name="s")

    @functools.partial(
        pl.kernel,
        out_type=jax.ShapeDtypeStruct((M * ROW,), jnp.float32),
        mesh=mesh,
        scratch_types=[
            pltpu.VMEM((QPT,), jnp.int32),
            pltpu.VMEM((QPT,), jnp.int32),
            pltpu.VMEM((QPT,), jnp.float32),
            pltpu.VMEM((QPT,), jnp.float32),
            pltpu.VMEM((QPT,), jnp.float32),
            pltpu.VMEM((8, 128), jnp.int32),
            pltpu.VMEM((8, 128), jnp.int32),
            pltpu.VMEM((8, 128), jnp.float32),
            pltpu.VMEM((8, 128), jnp.float32),
            pltpu.VMEM((8, 128), jnp.float32),
            pltpu.VMEM((80,), jnp.int32),
            pltpu.VMEM((80,), jnp.float32),
            pltpu.VMEM((80,), jnp.float32),
            pltpu.VMEM((80,), jnp.float32),
            pltpu.VMEM((1040,), jnp.int32),
            pltpu.VMEM((1040,), jnp.int32),
            pltpu.VMEM((1040,), jnp.int32),
            pltpu.VMEM((1040,), jnp.int32),
            pltpu.VMEM((144,), jnp.float32),
            pltpu.VMEM((2064,), jnp.float32),
            pltpu.VMEM((2064,), jnp.float32),
            pltpu.VMEM((2064,), jnp.float32),
            pltpu.VMEM((4, 32, 2 * C), jnp.float32),
            pltpu.VMEM((67 * 1024,), jnp.float32),
            pltpu.SemaphoreType.DMA,
            pltpu.SemaphoreType.DMA,
            pltpu.SemaphoreType.DMA,
            pltpu.SemaphoreType.DMA,
            pltpu.SemaphoreType.DMA,
        ],
        compiler_params=pltpu.CompilerParams(needs_layout_passes=False),
    )
    def k(xh_h, yh_h, zh_h, feat_h, rv_h, rows_h, cols_h, qx_h, qy_h, qz_h,
          out_h, rows_v, cols_v, qx_v, qy_v, qz_v, idxb, candb, cx, cy, cz,
          seli, selx, sely, selz, idsA, idsB, parA, parB, zfq,
          relx, rely, relz, featb, obuf, sem, fsem0, fsem1, fsem2, fsem3):
        wid = lax.axis_index("s") * 2 + lax.axis_index("c")
        base = wid * QPT
        pltpu.sync_copy(rows_h.at[pl.ds(base, QPT)], rows_v)
        pltpu.sync_copy(cols_h.at[pl.ds(base, QPT)], cols_v)
        pltpu.sync_copy(qx_h.at[pl.ds(base, QPT)], qx_v)
        pltpu.sync_copy(qy_h.at[pl.ds(base, QPT)], qy_v)
        pltpu.sync_copy(qz_h.at[pl.ds(base, QPT)], qz_v)

        iota = lax.iota(jnp.int32, 16)
        iota1024 = iota * 1024
        mA = iota < 8
        mB = iota >= 8
        fsems = [fsem0, fsem1, fsem2, fsem3]

        def window_rel(j):
            kv = iota + 16 * j
            hq = lax.shift_right_logical(kv * 94, 10)
            wq = kv - hq * _W_RANGE
            return hq - 2, wq - 5, kv

        def sel_query(qq, r, c, qxs, qys, qzs, q):
            cnt = jnp.int32(0)
            for j in range(_KPAD // 16):
                p = qq * _KPAD + j * 16
                rw, off = p // 128, p % 128
                cand = candb[rw, pl.ds(off, 16)]
                xs = cx[rw, pl.ds(off, 16)]
                ys = cy[rw, pl.ds(off, 16)]
                zs = cz[rw, pl.ds(off, 16)]
                hr, wr, kv = window_rel(j)
                rp = r + hr
                cp = c + wr
                ok = (rp >= 0) & (rp < _RH) & (cp >= 0) & (cp < _RW)
                if j == 3:
                    ok = ok & (kv < _H_RANGE * _W_RANGE)
                dx = xs - qxs
                dy = ys - qys
                dz = zs - qzs
                d2 = dx * dx + dy * dy + dz * dz
                inb = ok & (cand >= 0) & (d2 <= _RADIUS2)
                bi = jnp.where(inb, 1, 0)
                pos = plsc.cumsum(bi) - 1 + cnt
                plsc.store_scatter(seli, [pos], cand, mask=inb)
                plsc.store_scatter(selx, [pos], xs, mask=inb)
                plsc.store_scatter(sely, [pos], ys, mask=inb)
                plsc.store_scatter(selz, [pos], zs, mask=inb)
                cnt = cnt + jnp.sum(bi)

            nonempty = cnt > 0
            pos16 = jnp.where(iota < cnt, iota, 0)
            sid = plsc.load_gather(seli, [pos16])
            gx = plsc.load_gather(selx, [pos16])
            gy = plsc.load_gather(sely, [pos16])
            gz = plsc.load_gather(selz, [pos16])
            sid = jnp.where(nonempty, sid, 0)
            sidh = lax.shift_right_logical(sid, 1)
            sidp = sid & 1
            posA = q * 8 + iota
            plsc.store_scatter(idsA, [posA], sidh, mask=mA)
            plsc.store_scatter(idsB, [posA - 8], sidh, mask=mB)
            plsc.store_scatter(parA, [posA], sidp, mask=mA)
            plsc.store_scatter(parB, [posA - 8], sidp, mask=mB)
            zv = jnp.where(nonempty, jnp.float32(1.0), jnp.float32(0.0))
            zv = zv + iota * 0.0
            plsc.store_scatter(zfq, [iota * 0 + q], zv, mask=iota < 1)
            posr = q * 16 + iota
            plsc.store_scatter(relx, [posr],
                               jnp.where(nonempty, gx - qxs, 0.0))
            plsc.store_scatter(rely, [posr],
                               jnp.where(nonempty, gy - qys, 0.0))
            plsc.store_scatter(relz, [posr],
                               jnp.where(nonempty, gz - qzs, 0.0))

        def sel_group(mt, gi):
            start = mt * 128 + gi * G
            rows16 = rows_v[pl.ds(start, 16)]
            cols16 = cols_v[pl.ds(start, 16)]
            qx16 = qx_v[pl.ds(start, 16)]
            qy16 = qy_v[pl.ds(start, 16)]
            qz16 = qz_v[pl.ds(start, 16)]

            def sel_pair(rw):
                for qq in (2 * rw, 2 * rw + 1):
                    sel_query(qq, rows16[qq], cols16[qq],
                              qx16[qq], qy16[qq], qz16[qq], gi * G + qq)

            cand_cps = []
            for rw in range(8):
                for qq in (2 * rw, 2 * rw + 1):
                    r = rows16[qq]
                    c = cols16[qq]
                    for j in range(_KPAD // 16):
                        hr, wr, _ = window_rel(j)
                        rp = r + hr
                        cp = c + wr
                        rc = jnp.clip(rp, 0, _RH - 1)
                        cc = jnp.clip(cp, 0, _RW - 1)
                        p = qq * _KPAD + j * 16
                        idxb[p // 128, pl.ds(p % 128, 16)] = rc * _RW + cc
                cand_cps.append(pltpu.async_copy(
                    rv_h.at[idxb.at[rw]], candb.at[rw], sem))
            xyz_cps = {}
            for rw in range(8):
                cand_cps[rw].wait()
                xyz_cps[rw] = (
                    pltpu.async_copy(xh_h.at[candb.at[rw]], cx.at[rw],
                                     fsem0),
                    pltpu.async_copy(yh_h.at[candb.at[rw]], cy.at[rw],
                                     fsem0),
                    pltpu.async_copy(zh_h.at[candb.at[rw]], cz.at[rw],
                                     fsem0),
                )
                if rw >= 2:
                    for cp in xyz_cps[rw - 2]:
                        cp.wait()
                    sel_pair(rw - 2)
            for rw in (6, 7):
                for cp in xyz_cps[rw]:
                    cp.wait()
                sel_pair(rw)

        def transpose_query(qq, buf, q, st, par, zq):
            parv = par[pl.ds(q * 8, 16)]
            rvx = relx[pl.ds(q * 16, 16)]
            rvy = rely[pl.ds(q * 16, 16)]
            rvz = relz[pl.ds(q * 16, 16)]
            if st == 0:
                mask, s8v = mA, iota
            else:
                mask, s8v = mB, iota - 8
            posr = s8v * 128 + q
            plsc.store_scatter(obuf, [posr], rvx, mask=mask)
            plsc.store_scatter(obuf, [posr + 1024], rvy, mask=mask)
            plsc.store_scatter(obuf, [posr + 2048], rvz, mask=mask)
            for s8 in range(8):
                off = pl.multiple_of(parv[s8] * C, C)
                frow = (qq % 4) * 8 + s8
                sbase = 3 * 1024 + s8 * 128 + q
                for cb in range(C // 16):
                    vec = featb[buf, frow, pl.ds(off + cb * 16, 16)]
                    posv = iota1024 + (sbase + cb * 16 * 1024)
                    plsc.store_scatter(obuf, [posv], vec * zq)

        def write_pass(mt, st):
            ids = idsA if st == 0 else idsB
            par = parA if st == 0 else parB

            def gi_body(gi, carry):
                gbase = gi * 128
                cp = pltpu.async_copy(
                    feat_h.at[ids.at[pl.ds(gbase, 32)]], featb.at[0],
                    fsems[0])
                for h in range(4):
                    nxt = None
                    if h < 3:
                        nxt = pltpu.async_copy(
                            feat_h.at[ids.at[pl.ds(gbase + (h + 1) * 32,
                                                   32)]],
                            featb.at[(h + 1) % 2], fsems[(h + 1) % 2])
                    cp.wait()
                    zv16 = zfq[pl.ds(gi * 16, 16)]
                    for qq in range(h * 4, h * 4 + 4):
                        transpose_query(qq, (h % 2), gi * G + qq, st, par,
                                        zv16[qq])
                    cp = nxt
                return carry

            lax.fori_loop(0, 128 // G, gi_body, jnp.int32(0))
            mtg = wid * NMT + mt
            cps = [pltpu.async_copy(
                obuf.at[pl.ds(ch * 1024, 1024)],
                out_h.at[pl.ds(((2 * ch + st) * 128 + mtg) * 1024, 1024)],
                sem)
                for ch in range(CO)]
            for cp in cps:
                cp.wait()

        def mt_body(mt, carry):
            def sel_gi(gi, c):
                sel_group(mt, gi)
                return c

            lax.fori_loop(0, 128 // G, sel_gi, jnp.int32(0))
            write_pass(mt, 0)
            write_pass(mt, 1)
            return carry

        lax.fori_loop(0, NMT, mt_body, jnp.int32(0))

    return k(xh, yh, zh, features, rv_flat, rows, cols, qx, qy, qz)


def _pair_rows_tc(features):
    N, C = features.shape
    BR = 8000
    grid = N // BR

    def body(x_ref, o_ref):
        o_ref[:, :C] = x_ref[0::2, :]
        o_ref[:, C:] = x_ref[1::2, :]

    return pl.pallas_call(
        body,
        grid=(grid,),
        in_specs=[pl.BlockSpec((BR, C), lambda i: (i, 0))],
        out_specs=pl.BlockSpec((BR // 2, 2 * C), lambda i: (i, 0)),
        out_shape=jax.ShapeDtypeStruct((N // 2, 2 * C), features.dtype),
    )(features)


def kernel(xyz, features, query_rv_xyz, query_rv_coords, rv_map):
    M = query_rv_xyz.shape[0]
    C = features.shape[1]
    features = _pair_rows_tc(features)
    rv_flat = rv_map.reshape(-1)
    rows = query_rv_coords[:, 1].astype(jnp.int32)
    cols = query_rv_coords[:, 2].astype(jnp.int32)
    xh = xyz[:, 0]
    yh = xyz[:, 1]
    zh = xyz[:, 2]
    qx = query_rv_xyz[:, 0]
    qy = query_rv_xyz[:, 1]
    qz = query_rv_xyz[:, 2]
    out = _sc_kernel(xh, yh, zh, features, rv_flat, rows, cols, qx, qy, qz)
    out = out.reshape(3 + C, 2, M // 128, 8, 128)
    out = out.transpose(2, 4, 0, 1, 3)
    return out.reshape(M, 3 + C, _NSAMPLE)
